# Initial kernel scaffold; baseline (speedup 1.0000x reference)
import jax
import jax.numpy as jnp
from jax.experimental import pallas as pl


def kernel(samples, matches, refs):
    raise NotImplementedError("write your pallas kernel here")


# speedup vs baseline: 2.9632x; 2.9632x over previous
import functools

import jax
import jax.numpy as jnp
from jax import lax
from jax.experimental import pallas as pl
from jax.experimental.pallas import tpu as pltpu
from jax.experimental.pallas import tpu_sc as plsc

_B, _N, _M = 16, 20000, 100
_NC, _NS, _L = 2, 16, 16
_NW = _NC * _NS
_CHUNK = (_B * _N) // _NW
_STEPS = _CHUNK // _L

_mesh = plsc.VectorSubcoreMesh(core_axis_name="c", subcore_axis_name="s")


@functools.partial(
    pl.kernel,
    mesh=_mesh,
    out_type=jax.ShapeDtypeStruct((_B * _N,), jnp.int32),
    compiler_params=pltpu.CompilerParams(needs_layout_passes=False),
    scratch_types=[
        pltpu.VMEM((_CHUNK,), jnp.float32),
        pltpu.VMEM((_CHUNK,), jnp.int32),
        pltpu.VMEM((_B * _M,), jnp.int32),
        pltpu.VMEM((_CHUNK,), jnp.int32),
    ],
)
def _encode(samples_hbm, matches_hbm, refs_hbm, out_hbm, s_v, m_v, r_v, o_v):
    wid = lax.axis_index("s") * _NC + lax.axis_index("c")
    base = wid * _CHUNK
    row_off = (wid // 2) * _M

    pltpu.sync_copy(samples_hbm.at[pl.ds(base, _CHUNK)], s_v)
    pltpu.sync_copy(matches_hbm.at[pl.ds(base, _CHUNK)], m_v)
    pltpu.sync_copy(refs_hbm, r_v)

    def body(i, _):
        sl = pl.ds(i * _L, _L)
        idx = jnp.clip(m_v[sl], 0, _M - 1) + row_off
        t = plsc.load_gather(r_v, [idx]) + 1
        s = s_v[sl]
        o_v[sl] = jnp.where(s > 0.5, t,
                            jnp.where(s < -0.5,
                                      jnp.zeros_like(t),
                                      jnp.full_like(t, -1)))
        return ()

    lax.fori_loop(0, _STEPS, body, ())
    pltpu.sync_copy(o_v, out_hbm.at[pl.ds(base, _CHUNK)])


def kernel(samples, matches, refs):
    matches = matches.astype(jnp.int32)
    refs = refs.astype(jnp.int32)
    flat = _encode(samples.reshape(-1), matches.reshape(-1), refs.reshape(-1))
    return flat.reshape(_B, _N)


# speedup vs baseline: 3.4157x; 1.1527x over previous
import functools

import jax
import jax.numpy as jnp
from jax import lax
from jax.experimental import pallas as pl
from jax.experimental.pallas import tpu as pltpu
from jax.experimental.pallas import tpu_sc as plsc

_B, _N, _M = 16, 20000, 100
_MP = 104
_NC, _NS, _L = 2, 16, 16
_NW = _NC * _NS
_CHUNK = (_B * _N) // _NW

_mesh = plsc.VectorSubcoreMesh(core_axis_name="c", subcore_axis_name="s")


@functools.partial(
    pl.kernel,
    mesh=_mesh,
    out_type=jax.ShapeDtypeStruct((_B * _N,), jnp.int32),
    compiler_params=pltpu.CompilerParams(needs_layout_passes=False),
    scratch_types=[
        pltpu.VMEM((_CHUNK,), jnp.float32),
        pltpu.VMEM((_CHUNK,), jnp.int32),
        pltpu.VMEM((_MP,), jnp.int32),
        pltpu.VMEM((_CHUNK,), jnp.int32),
        pltpu.SemaphoreType.DMA,
        pltpu.SemaphoreType.DMA,
        pltpu.SemaphoreType.DMA,
    ],
)
def _encode(samples_hbm, matches_hbm, refs_hbm, out_hbm,
            s_v, m_v, r_v, o_v, sem_s, sem_m, sem_r):
    wid = lax.axis_index("s") * _NC + lax.axis_index("c")
    base = wid * _CHUNK
    b = wid // 2

    cp_s = pltpu.async_copy(samples_hbm.at[pl.ds(base, _CHUNK)], s_v, sem_s)
    cp_m = pltpu.async_copy(matches_hbm.at[pl.ds(base, _CHUNK)], m_v, sem_m)
    cp_r = pltpu.async_copy(refs_hbm.at[b], r_v, sem_r)
    cp_r.wait()
    cp_m.wait()
    cp_s.wait()

    @plsc.parallel_loop(0, _CHUNK, _L, unroll=8)
    def _body(i):
        sl = pl.ds(i, _L)
        t = plsc.load_gather(r_v, [m_v[sl]]) + 1
        s = s_v[sl]
        o_v[sl] = jnp.where(s > 0.5, t,
                            jnp.where(s < -0.5,
                                      jnp.zeros_like(t),
                                      jnp.full_like(t, -1)))

    pltpu.sync_copy(o_v, out_hbm.at[pl.ds(base, _CHUNK)])


def kernel(samples, matches, refs):
    matches = matches.astype(jnp.int32)
    refs = jnp.pad(refs.astype(jnp.int32), ((0, 0), (0, _MP - _M)))
    flat = _encode(samples.reshape(-1), matches.reshape(-1), refs)
    return flat.reshape(_B, _N)


# speedup vs baseline: 3.8989x; 1.1415x over previous
import functools

import jax
import jax.numpy as jnp
from jax import lax
from jax.experimental import pallas as pl
from jax.experimental.pallas import tpu as pltpu
from jax.experimental.pallas import tpu_sc as plsc

_B, _N, _M = 16, 20000, 100
_NC, _NS, _L = 2, 16, 16
_NW = _NC * _NS
_WCOLS = 1280
_WCOLS_LAST = 896
_ROWS = 8

_mesh = plsc.VectorSubcoreMesh(core_axis_name="c", subcore_axis_name="s")


@functools.partial(
    pl.kernel,
    mesh=_mesh,
    out_type=jax.ShapeDtypeStruct((_B, _N), jnp.int32),
    compiler_params=pltpu.CompilerParams(needs_layout_passes=False),
    scratch_types=[
        pltpu.VMEM((_ROWS, _WCOLS), jnp.float32),
        pltpu.VMEM((_ROWS, _WCOLS), jnp.int32),
        pltpu.VMEM((_B, _M), jnp.int32),
        pltpu.VMEM((_ROWS, _WCOLS), jnp.int32),
        pltpu.SemaphoreType.DMA,
        pltpu.SemaphoreType.DMA,
        pltpu.SemaphoreType.DMA,
    ],
)
def _encode(samples_hbm, matches_hbm, refs_hbm, out_hbm,
            s_v, m_v, r_v, o_v, sem_s, sem_m, sem_r):
    wid = lax.axis_index("s") * _NC + lax.axis_index("c")
    band = wid // 16
    col_w = wid % 16
    r0 = band * _ROWS
    c0 = col_w * _WCOLS
    is_last = col_w == 15
    ncols = jnp.where(is_last, _WCOLS_LAST, _WCOLS)

    cp_r = pltpu.async_copy(refs_hbm, r_v, sem_r)

    @pl.when(is_last)
    def _():
        pltpu.async_copy(
            samples_hbm.at[pl.ds(r0, _ROWS), pl.ds(c0, _WCOLS_LAST)],
            s_v.at[:, pl.ds(0, _WCOLS_LAST)], sem_s).wait()
        pltpu.async_copy(
            matches_hbm.at[pl.ds(r0, _ROWS), pl.ds(c0, _WCOLS_LAST)],
            m_v.at[:, pl.ds(0, _WCOLS_LAST)], sem_m).wait()

    @pl.when(jnp.logical_not(is_last))
    def _():
        pltpu.async_copy(
            samples_hbm.at[pl.ds(r0, _ROWS), pl.ds(c0, _WCOLS)],
            s_v, sem_s).wait()
        pltpu.async_copy(
            matches_hbm.at[pl.ds(r0, _ROWS), pl.ds(c0, _WCOLS)],
            m_v, sem_m).wait()

    cp_r.wait()

    for r in range(_ROWS):
        b_vec = jnp.full((_L,), r0 + r, jnp.int32)

        @plsc.parallel_loop(0, ncols, _L, unroll=8)
        def _body(i):
            sl = pl.ds(i, _L)
            mi = jnp.clip(m_v[r, sl], 0, _M - 1)
            t = plsc.load_gather(r_v, [b_vec, mi]) + 1
            s = s_v[r, sl]
            o_v[r, sl] = jnp.where(s > 0.5, t,
                                   jnp.where(s < -0.5,
                                             jnp.zeros_like(t),
                                             jnp.full_like(t, -1)))

    @pl.when(is_last)
    def _():
        pltpu.sync_copy(o_v.at[:, pl.ds(0, _WCOLS_LAST)],
                        out_hbm.at[pl.ds(r0, _ROWS), pl.ds(c0, _WCOLS_LAST)])

    @pl.when(jnp.logical_not(is_last))
    def _():
        pltpu.sync_copy(o_v, out_hbm.at[pl.ds(r0, _ROWS), pl.ds(c0, _WCOLS)])


def kernel(samples, matches, refs):
    return _encode(samples, matches.astype(jnp.int32), refs.astype(jnp.int32))


# speedup vs baseline: 4.1356x; 1.0607x over previous
import functools

import jax
import jax.numpy as jnp
from jax import lax
from jax.experimental import pallas as pl
from jax.experimental.pallas import tpu as pltpu
from jax.experimental.pallas import tpu_sc as plsc

_B, _N, _M = 16, 20000, 100
_L = 16
_TILES = 10
_TILES_LAST = 7
_ROWS = 8

_mesh = plsc.VectorSubcoreMesh(core_axis_name="c", subcore_axis_name="s")


@functools.partial(
    pl.kernel,
    mesh=_mesh,
    out_type=jax.ShapeDtypeStruct((_B, _N), jnp.int32),
    compiler_params=pltpu.CompilerParams(needs_layout_passes=False),
    scratch_types=[
        pltpu.VMEM((_TILES, _ROWS, 128), jnp.float32),
        pltpu.VMEM((_TILES, _ROWS, 128), jnp.int32),
        pltpu.VMEM((_B, _M), jnp.int32),
        pltpu.VMEM((_TILES, _ROWS, 128), jnp.int32),
        pltpu.SemaphoreType.DMA,
        pltpu.SemaphoreType.DMA,
        pltpu.SemaphoreType.DMA,
        pltpu.SemaphoreType.DMA,
    ],
)
def _encode(samples_hbm, matches_hbm, refs_hbm, out_hbm,
            s_v, m_v, r_v, o_v, sem_s, sem_m, sem_r, sem_o):
    wid = lax.axis_index("s") * 2 + lax.axis_index("c")
    band = wid // 16
    col_w = wid % 16
    r0 = band * _ROWS
    c0 = col_w * (_TILES * 128)
    ntiles = jnp.where(col_w == 15, _TILES_LAST, _TILES)

    cp_r = pltpu.async_copy(refs_hbm, r_v, sem_r)

    def _issue(j, _):
        src = pl.ds(c0 + j * 128, 128)
        pltpu.async_copy(samples_hbm.at[pl.ds(r0, _ROWS), src], s_v.at[j], sem_s)
        pltpu.async_copy(matches_hbm.at[pl.ds(r0, _ROWS), src], m_v.at[j], sem_m)
        return ()

    lax.fori_loop(0, ntiles, _issue, ())
    cp_r.wait()

    def _tile(j, _):
        pltpu.make_async_copy(
            samples_hbm.at[pl.ds(r0, _ROWS), pl.ds(c0, 128)], s_v.at[j], sem_s
        ).wait()
        pltpu.make_async_copy(
            matches_hbm.at[pl.ds(r0, _ROWS), pl.ds(c0, 128)], m_v.at[j], sem_m
        ).wait()

        @plsc.parallel_loop(0, _ROWS * 128, _L, unroll=4)
        def _body(i):
            r = i >> 7
            sl = pl.ds(i & 127, _L)
            b_vec = jnp.full((_L,), r0 + r, jnp.int32)
            mi = jnp.clip(m_v[j, r, sl], 0, _M - 1)
            t = plsc.load_gather(r_v, [b_vec, mi]) + 1
            s = s_v[j, r, sl]
            o_v[j, r, sl] = jnp.where(s > 0.5, t,
                                      jnp.where(s < -0.5,
                                                jnp.zeros_like(t),
                                                jnp.full_like(t, -1)))

        pltpu.async_copy(
            o_v.at[j], out_hbm.at[pl.ds(r0, _ROWS), pl.ds(c0 + j * 128, 128)],
            sem_o)
        return ()

    lax.fori_loop(0, ntiles, _tile, ())

    def _drain(j, _):
        pltpu.make_async_copy(
            o_v.at[j], out_hbm.at[pl.ds(r0, _ROWS), pl.ds(c0, 128)], sem_o
        ).wait()
        return ()

    lax.fori_loop(0, ntiles, _drain, ())


def kernel(samples, matches, refs):
    return _encode(samples, matches.astype(jnp.int32), refs.astype(jnp.int32))
